# baseline (speedup 1.0000x reference)
import functools

import jax
import jax.numpy as jnp
from jax.experimental import pallas as pl
from jax.experimental.pallas import tpu as pltpu

_BM = 400
_NBUF = 3


def _gcn_kernel(adj_hbm, x_ref, w1_ref, b1_ref, w2_ref, b2_ref, out_hbm,
                ring, h_ref, stage, sems, out_sems, *, nb):
    total = 2 * nb

    def _start(s, b):
        r = jax.lax.rem(s, nb)
        pltpu.make_async_copy(
            adj_hbm.at[pl.ds(r * _BM, _BM), :], ring.at[b], sems.at[b]
        ).start()

    for s in range(_NBUF):
        _start(s, s)

    def _step(s, carry):
        r = jax.lax.rem(s, nb)

        def _make_branch(b):
            def _branch():
                pltpu.make_async_copy(
                    adj_hbm.at[pl.ds(r * _BM, _BM), :], ring.at[b], sems.at[b]
                ).wait()

                @pl.when(s < nb)
                def _layer1():
                    g = jnp.dot(ring[b], x_ref[...],
                                preferred_element_type=jnp.float32)
                    h = jnp.dot(g, w1_ref[...],
                                preferred_element_type=jnp.float32) + b1_ref[...]
                    h_ref[pl.ds(r * _BM, _BM), :] = jnp.maximum(h, 0.0)

                @pl.when(s >= nb)
                def _layer2():
                    @pl.when(s >= nb + _NBUF)
                    def _wait_prev():
                        pltpu.make_async_copy(
                            stage.at[b], out_hbm.at[pl.ds(r * _BM, _BM), :],
                            out_sems.at[b]
                        ).wait()

                    g = jnp.dot(ring[b], h_ref[...],
                                preferred_element_type=jnp.float32)
                    stage[b] = jnp.dot(
                        g, w2_ref[...],
                        preferred_element_type=jnp.float32) + b2_ref[...]
                    pltpu.make_async_copy(
                        stage.at[b], out_hbm.at[pl.ds(r * _BM, _BM), :],
                        out_sems.at[b]
                    ).start()

                @pl.when(s + _NBUF < total)
                def _prefetch():
                    _start(s + _NBUF, b)

            return _branch

        jax.lax.switch(jax.lax.rem(s, _NBUF),
                       [_make_branch(b) for b in range(_NBUF)])
        return carry

    jax.lax.fori_loop(0, total, _step, 0)

    for b in range(_NBUF):
        pltpu.make_async_copy(
            stage.at[b], out_hbm.at[pl.ds(0, _BM), :], out_sems.at[b]
        ).wait()


def kernel(x, adj, W1, b1, W2, b2):
    n, d = x.shape
    nb = n // _BM
    return pl.pallas_call(
        functools.partial(_gcn_kernel, nb=nb),
        in_specs=[
            pl.BlockSpec(memory_space=pl.ANY),
            pl.BlockSpec(memory_space=pltpu.MemorySpace.VMEM),
            pl.BlockSpec(memory_space=pltpu.MemorySpace.VMEM),
            pl.BlockSpec(memory_space=pltpu.MemorySpace.VMEM),
            pl.BlockSpec(memory_space=pltpu.MemorySpace.VMEM),
            pl.BlockSpec(memory_space=pltpu.MemorySpace.VMEM),
        ],
        out_specs=pl.BlockSpec(memory_space=pl.ANY),
        out_shape=jax.ShapeDtypeStruct((n, d), jnp.float32),
        scratch_shapes=[
            pltpu.VMEM((_NBUF, _BM, n), jnp.float32),
            pltpu.VMEM((n, d), jnp.float32),
            pltpu.VMEM((_NBUF, _BM, d), jnp.float32),
            pltpu.SemaphoreType.DMA((_NBUF,)),
            pltpu.SemaphoreType.DMA((_NBUF,)),
        ],
    )(adj, x, W1, b1.reshape(1, -1), W2, b2.reshape(1, -1))
